# Initial kernel scaffold; baseline (speedup 1.0000x reference)
import functools

import jax
import jax.numpy as jnp
from jax import lax
from jax.experimental import pallas as pl
from jax.experimental.pallas import tpu as pltpu
from jax.experimental.pallas import tpu_sc as plsc

NC = 2
NS = 16
NT = NC * NS
CHUNK = 128
DEG_W = 16


def _sc_mesh():
    return plsc.VectorSubcoreMesh(core_axis_name="c", subcore_axis_name="s",
                                  num_cores=NC, num_subcores=NS)


def _deg_kernel(n_pad, n_chunks):
    rows_per_tile = n_pad // NS
    nz = rows_per_tile // 16

    @functools.partial(
        pl.kernel,
        out_type=jax.ShapeDtypeStruct((NC, n_pad, DEG_W), jnp.float32),
        mesh=_sc_mesh(),
        scratch_types=[
            pltpu.VMEM((n_chunks, CHUNK), jnp.int32),
            pltpu.VMEM((CHUNK, DEG_W), jnp.float32),
            pltpu.VMEM((16, DEG_W), jnp.float32),
            pltpu.VMEM_SHARED((n_pad, DEG_W), jnp.float32),
        ],
    )
    def deg_kernel(dst_hbm, out_hbm, idx_v, ones_v, z_v, acc_sh):
        c = lax.axis_index("c")
        s = lax.axis_index("s")
        wid = c * NS + s
        pltpu.sync_copy(dst_hbm.at[wid], idx_v)

        one = jnp.ones((16,), jnp.float32)
        zero = jnp.zeros((16,), jnp.float32)

        def fill_ones(r, carry):
            ones_v[r, :] = one
            return carry
        lax.fori_loop(0, CHUNK, fill_ones, 0)

        def fill_zero(r, carry):
            z_v[r, :] = zero
            return carry
        lax.fori_loop(0, 16, fill_zero, 0)

        base = s * rows_per_tile

        def zero_acc(i, carry):
            pltpu.sync_copy(z_v, acc_sh.at[pl.ds(base + i * 16, 16)])
            return carry
        lax.fori_loop(0, nz, zero_acc, 0)

        plsc.subcore_barrier()

        def body(j, carry):
            pltpu.sync_copy(ones_v, acc_sh.at[idx_v.at[j]], add=True)
            return carry
        lax.fori_loop(0, n_chunks, body, 0)

        plsc.subcore_barrier()
        pltpu.sync_copy(acc_sh.at[pl.ds(base, rows_per_tile)],
                        out_hbm.at[c, pl.ds(base, rows_per_tile)])

    return deg_kernel


def _agg_kernel(n_pad, n_chunks, d):
    rows_per_tile = n_pad // NS
    nz = rows_per_tile // 16

    @functools.partial(
        pl.kernel,
        out_type=jax.ShapeDtypeStruct((NC, n_pad, d), jnp.float32),
        mesh=_sc_mesh(),
        scratch_types=[
            pltpu.VMEM((n_chunks, CHUNK), jnp.int32),
            pltpu.VMEM((n_chunks, CHUNK), jnp.int32),
            pltpu.VMEM((CHUNK, d), jnp.float32),
            pltpu.VMEM((CHUNK, d), jnp.float32),
            pltpu.VMEM((16, d), jnp.float32),
            pltpu.VMEM_SHARED((n_pad, d), jnp.float32),
            pltpu.SemaphoreType.DMA,
            pltpu.SemaphoreType.DMA,
        ],
    )
    def agg_kernel(x2_hbm, src_hbm, dst_hbm, out_hbm,
                   src_v, dst_v, buf0, buf1, z_v, acc_sh, sem0, sem1):
        c = lax.axis_index("c")
        s = lax.axis_index("s")
        wid = c * NS + s
        pltpu.sync_copy(src_hbm.at[wid], src_v)
        pltpu.sync_copy(dst_hbm.at[wid], dst_v)

        zero = jnp.zeros((16,), jnp.float32)

        def fill_zero(r, carry):
            for q in range(d // 16):
                z_v[r, pl.ds(q * 16, 16)] = zero
            return carry
        lax.fori_loop(0, 16, fill_zero, 0)

        base = s * rows_per_tile

        def zero_acc(i, carry):
            pltpu.sync_copy(z_v, acc_sh.at[pl.ds(base + i * 16, 16)])
            return carry
        lax.fori_loop(0, nz, zero_acc, 0)

        plsc.subcore_barrier()

        pltpu.async_copy(x2_hbm.at[src_v.at[0]], buf0, sem0)

        def body(jj, carry):
            j = jj * 2
            pltpu.async_copy(x2_hbm.at[src_v.at[j + 1]], buf1, sem1)
            pltpu.make_async_copy(x2_hbm.at[src_v.at[j]], buf0, sem0).wait()
            pltpu.sync_copy(buf0, acc_sh.at[dst_v.at[j]], add=True)

            @pl.when(jj < (n_chunks // 2) - 1)
            def _():
                pltpu.async_copy(x2_hbm.at[src_v.at[j + 2]], buf0, sem0)

            pltpu.make_async_copy(x2_hbm.at[src_v.at[j + 1]], buf1, sem1).wait()
            pltpu.sync_copy(buf1, acc_sh.at[dst_v.at[j + 1]], add=True)
            return carry
        lax.fori_loop(0, n_chunks // 2, body, 0)

        plsc.subcore_barrier()
        pltpu.sync_copy(acc_sh.at[pl.ds(base, rows_per_tile)],
                        out_hbm.at[c, pl.ds(base, rows_per_tile)])

    return agg_kernel


def _scale_body(deg_ref, x_ref, o_ref):
    dcol = deg_ref[0][:, :1] + deg_ref[1][:, :1]
    dis = jnp.where(dcol > 0, lax.rsqrt(jnp.where(dcol > 0, dcol, 1.0)), 0.0)
    o_ref[...] = x_ref[...] * dis


def _finish_body(acc_ref, deg_ref, w_ref, b_ref, o_ref):
    dcol = deg_ref[0][:, :1] + deg_ref[1][:, :1]
    dis = jnp.where(dcol > 0, lax.rsqrt(jnp.where(dcol > 0, dcol, 1.0)), 0.0)
    agg = acc_ref[0] + acc_ref[1]
    h = jnp.dot(agg, w_ref[...], preferred_element_type=jnp.float32,
                precision=lax.Precision.HIGHEST)
    o_ref[...] = dis * h + b_ref[...]


def kernel(x, edge_index, W, b):
    n, d_in = x.shape
    d_out = W.shape[1]
    e = edge_index.shape[1]

    n_pad = ((n + 1 + NS * 16 - 1) // (NS * 16)) * (NS * 16)
    e_per_tile = ((e + NT * CHUNK - 1) // (NT * CHUNK)) * CHUNK
    n_chunks = e_per_tile // CHUNK
    e_pad = e_per_tile * NT

    src = edge_index[0]
    dst = edge_index[1]
    src_p = jnp.concatenate(
        [src, jnp.zeros((e_pad - e,), jnp.int32)]).reshape(NT, n_chunks, CHUNK)
    dst_p = jnp.concatenate(
        [dst, jnp.full((e_pad - e,), n, jnp.int32)]).reshape(NT, n_chunks, CHUNK)

    deg_p = _deg_kernel(n_pad, n_chunks)(dst_p)

    blk = 1000 if n % 1000 == 0 else min(n, 1024)
    grid = (n + blk - 1) // blk

    x2 = pl.pallas_call(
        _scale_body,
        grid=(grid,),
        in_specs=[
            pl.BlockSpec((NC, blk, DEG_W), lambda i: (0, i, 0)),
            pl.BlockSpec((blk, d_in), lambda i: (i, 0)),
        ],
        out_specs=pl.BlockSpec((blk, d_in), lambda i: (i, 0)),
        out_shape=jax.ShapeDtypeStruct((n, d_in), jnp.float32),
    )(deg_p, x)

    acc_p = _agg_kernel(n_pad, n_chunks, d_in)(x2, src_p, dst_p)

    out = pl.pallas_call(
        _finish_body,
        grid=(grid,),
        in_specs=[
            pl.BlockSpec((NC, blk, d_in), lambda i: (0, i, 0)),
            pl.BlockSpec((NC, blk, DEG_W), lambda i: (0, i, 0)),
            pl.BlockSpec((d_in, d_out), lambda i: (0, 0)),
            pl.BlockSpec((d_out,), lambda i: (0,)),
        ],
        out_specs=pl.BlockSpec((blk, d_out), lambda i: (i, 0)),
        out_shape=jax.ShapeDtypeStruct((n, d_out), jnp.float32),
    )(acc_p, deg_p, W, b)

    return out


# speedup vs baseline: 11.4864x; 11.4864x over previous
import functools

import jax
import jax.numpy as jnp
from jax import lax
from jax.experimental import pallas as pl
from jax.experimental.pallas import tpu as pltpu
from jax.experimental.pallas import tpu_sc as plsc

NC = 2
NS = 16
NT = NC * NS
CHUNK = 64
ZROWS = 8
SHIFT = 14
MASK = (1 << SHIFT) - 1
NSEC = 4


def _sc_mesh():
    return plsc.VectorSubcoreMesh(core_axis_name="c", subcore_axis_name="s",
                                  num_cores=NC, num_subcores=NS)


def _deg_kernel(n_pad, n_chunks, d):
    rows_per_tile = n_pad // NS
    nz = rows_per_tile // ZROWS

    @functools.partial(
        pl.kernel,
        out_type=jax.ShapeDtypeStruct((NC, n_pad, d), jnp.float32),
        mesh=_sc_mesh(),
        scratch_types=[
            pltpu.VMEM((n_chunks, CHUNK), jnp.int32),
            pltpu.VMEM((CHUNK, d), jnp.float32),
            pltpu.VMEM((ZROWS, d), jnp.float32),
            pltpu.VMEM_SHARED((n_pad, d), jnp.float32),
        ],
    )
    def deg_kernel(pk_hbm, out_hbm, idx_v, ones_v, z_v, acc_sh):
        c = lax.axis_index("c")
        s = lax.axis_index("s")
        wid = c * NS + s
        pltpu.sync_copy(pk_hbm.at[wid], idx_v)

        def unpack(j, carry):
            for q in range(CHUNK // 16):
                v = idx_v[j, pl.ds(q * 16, 16)]
                idx_v[j, pl.ds(q * 16, 16)] = lax.shift_right_logical(v, SHIFT)
            return carry
        lax.fori_loop(0, n_chunks, unpack, 0)

        one = jnp.ones((16,), jnp.float32)
        zero = jnp.zeros((16,), jnp.float32)

        def fill_ones(r, carry):
            for q in range(d // 16):
                ones_v[r, pl.ds(q * 16, 16)] = one
            return carry
        lax.fori_loop(0, CHUNK, fill_ones, 0)

        def fill_zero(r, carry):
            for q in range(d // 16):
                z_v[r, pl.ds(q * 16, 16)] = zero
            return carry
        lax.fori_loop(0, ZROWS, fill_zero, 0)

        base = s * rows_per_tile

        def zero_acc(i, carry):
            pltpu.sync_copy(z_v, acc_sh.at[pl.ds(base + i * ZROWS, ZROWS)])
            return carry
        lax.fori_loop(0, nz, zero_acc, 0)

        plsc.subcore_barrier()

        def body(j, carry):
            pltpu.sync_copy(ones_v, acc_sh.at[idx_v.at[j]], add=True)
            return carry
        lax.fori_loop(0, n_chunks, body, 0)

        plsc.subcore_barrier()
        pltpu.sync_copy(acc_sh.at[pl.ds(base, rows_per_tile)],
                        out_hbm.at[c, pl.ds(base, rows_per_tile)])

    return deg_kernel


def _agg_kernel(n_pad, n_chunks, d):
    rows_per_tile = n_pad // NS
    nz = rows_per_tile // ZROWS
    sec_ch = n_chunks // NSEC

    @functools.partial(
        pl.kernel,
        out_type=jax.ShapeDtypeStruct((NC, n_pad, d), jnp.float32),
        mesh=_sc_mesh(),
        scratch_types=[
            pltpu.VMEM((sec_ch, CHUNK), jnp.int32),
            pltpu.VMEM((sec_ch, CHUNK), jnp.int32),
            pltpu.VMEM((CHUNK, d), jnp.float32),
            pltpu.VMEM((CHUNK, d), jnp.float32),
            pltpu.VMEM((ZROWS, d), jnp.float32),
            pltpu.VMEM_SHARED((n_pad, d), jnp.float32),
            pltpu.SemaphoreType.DMA,
            pltpu.SemaphoreType.DMA,
        ],
    )
    def agg_kernel(x2_hbm, pk_hbm, out_hbm,
                   src_v, dst_v, buf0, buf1, z_v, acc_sh, sem0, sem1):
        c = lax.axis_index("c")
        s = lax.axis_index("s")
        wid = c * NS + s

        zero = jnp.zeros((16,), jnp.float32)

        def fill_zero(r, carry):
            for q in range(d // 16):
                z_v[r, pl.ds(q * 16, 16)] = zero
            return carry
        lax.fori_loop(0, ZROWS, fill_zero, 0)

        base = s * rows_per_tile

        def zero_acc(i, carry):
            pltpu.sync_copy(z_v, acc_sh.at[pl.ds(base + i * ZROWS, ZROWS)])
            return carry
        lax.fori_loop(0, nz, zero_acc, 0)

        plsc.subcore_barrier()

        def section(sec, carry):
            pltpu.sync_copy(pk_hbm.at[wid, pl.ds(sec * sec_ch, sec_ch)], src_v)

            def unpack(j, carry2):
                for q in range(CHUNK // 16):
                    v = src_v[j, pl.ds(q * 16, 16)]
                    dst_v[j, pl.ds(q * 16, 16)] = lax.shift_right_logical(
                        v, SHIFT)
                    src_v[j, pl.ds(q * 16, 16)] = lax.bitwise_and(v, MASK)
                return carry2
            lax.fori_loop(0, sec_ch, unpack, 0)

            pltpu.async_copy(x2_hbm.at[src_v.at[0]], buf0, sem0)

            def body(jj, carry2):
                j = jj * 2
                pltpu.async_copy(x2_hbm.at[src_v.at[j + 1]], buf1, sem1)
                pltpu.make_async_copy(x2_hbm.at[src_v.at[j]], buf0,
                                      sem0).wait()
                pltpu.sync_copy(buf0, acc_sh.at[dst_v.at[j]], add=True)

                @pl.when(jj < (sec_ch // 2) - 1)
                def _():
                    pltpu.async_copy(x2_hbm.at[src_v.at[j + 2]], buf0, sem0)

                pltpu.make_async_copy(x2_hbm.at[src_v.at[j + 1]], buf1,
                                      sem1).wait()
                pltpu.sync_copy(buf1, acc_sh.at[dst_v.at[j + 1]], add=True)
                return carry2
            lax.fori_loop(0, sec_ch // 2, body, 0)
            return carry
        lax.fori_loop(0, NSEC, section, 0)

        plsc.subcore_barrier()
        pltpu.sync_copy(acc_sh.at[pl.ds(base, rows_per_tile)],
                        out_hbm.at[c, pl.ds(base, rows_per_tile)])

    return agg_kernel


def _scale_body(deg_ref, x_ref, o_ref):
    dcol = deg_ref[0][:, :1] + deg_ref[1][:, :1]
    dis = jnp.where(dcol > 0, lax.rsqrt(jnp.where(dcol > 0, dcol, 1.0)), 0.0)
    o_ref[...] = x_ref[...] * dis


def _finish_body(acc_ref, deg_ref, w_ref, b_ref, o_ref):
    dcol = deg_ref[0][:, :1] + deg_ref[1][:, :1]
    dis = jnp.where(dcol > 0, lax.rsqrt(jnp.where(dcol > 0, dcol, 1.0)), 0.0)
    agg = acc_ref[0] + acc_ref[1]
    h = jnp.dot(agg, w_ref[...], preferred_element_type=jnp.float32,
                precision=lax.Precision.HIGHEST)
    o_ref[...] = dis * h + b_ref[...]


def kernel(x, edge_index, W, b):
    n, d_in = x.shape
    d_out = W.shape[1]
    e = edge_index.shape[1]

    n_pad = ((n + 1 + NS * ZROWS - 1) // (NS * ZROWS)) * (NS * ZROWS)
    cm = CHUNK * NSEC * 2
    e_per_tile = ((e + NT * cm - 1) // (NT * cm)) * cm
    n_chunks = e_per_tile // CHUNK
    e_pad = e_per_tile * NT

    src = edge_index[0]
    dst = edge_index[1]
    pk = jnp.bitwise_or(jnp.left_shift(dst, SHIFT), src)
    pk_p = jnp.concatenate(
        [pk, jnp.full((e_pad - e,), n << SHIFT, jnp.int32)]
    ).reshape(NT, n_chunks, CHUNK)

    deg_p = _deg_kernel(n_pad, n_chunks, d_in)(pk_p)

    blk = 1000 if n % 1000 == 0 else min(n, 1024)
    grid = (n + blk - 1) // blk

    x2 = pl.pallas_call(
        _scale_body,
        grid=(grid,),
        in_specs=[
            pl.BlockSpec((NC, blk, d_in), lambda i: (0, i, 0)),
            pl.BlockSpec((blk, d_in), lambda i: (i, 0)),
        ],
        out_specs=pl.BlockSpec((blk, d_in), lambda i: (i, 0)),
        out_shape=jax.ShapeDtypeStruct((n, d_in), jnp.float32),
    )(deg_p, x)

    acc_p = _agg_kernel(n_pad, n_chunks, d_in)(x2, pk_p)

    out = pl.pallas_call(
        _finish_body,
        grid=(grid,),
        in_specs=[
            pl.BlockSpec((NC, blk, d_in), lambda i: (0, i, 0)),
            pl.BlockSpec((NC, blk, d_in), lambda i: (0, i, 0)),
            pl.BlockSpec((d_in, d_out), lambda i: (0, 0)),
            pl.BlockSpec((d_out,), lambda i: (0,)),
        ],
        out_specs=pl.BlockSpec((blk, d_out), lambda i: (i, 0)),
        out_shape=jax.ShapeDtypeStruct((n, d_out), jnp.float32),
    )(acc_p, deg_p, W, b)

    return out


# speedup vs baseline: 12.2470x; 1.0662x over previous
import functools

import jax
import jax.numpy as jnp
from jax import lax
from jax.experimental import pallas as pl
from jax.experimental.pallas import tpu as pltpu
from jax.experimental.pallas import tpu_sc as plsc

NC = 2
NS = 16
NT = NC * NS
CHUNK = 64
ZROWS = 8
SHIFT = 14
MASK = (1 << SHIFT) - 1
NSEC = 4


def _sc_mesh():
    return plsc.VectorSubcoreMesh(core_axis_name="c", subcore_axis_name="s",
                                  num_cores=NC, num_subcores=NS)


def _deg_kernel(n_pad, n_chunks, d):
    rows_per_tile = n_pad // NS
    nz = rows_per_tile // ZROWS

    @functools.partial(
        pl.kernel,
        out_type=jax.ShapeDtypeStruct((NC, n_pad, d), jnp.float32),
        mesh=_sc_mesh(),
        scratch_types=[
            pltpu.VMEM((n_chunks, CHUNK), jnp.int32),
            pltpu.VMEM((CHUNK, d), jnp.float32),
            pltpu.VMEM((ZROWS, d), jnp.float32),
            pltpu.VMEM_SHARED((n_pad, d), jnp.float32),
        ],
    )
    def deg_kernel(pk_hbm, out_hbm, idx_v, ones_v, z_v, acc_sh):
        c = lax.axis_index("c")
        s = lax.axis_index("s")
        wid = c * NS + s
        pltpu.sync_copy(pk_hbm.at[wid], idx_v)

        def unpack(j, carry):
            for q in range(CHUNK // 16):
                v = idx_v[j, pl.ds(q * 16, 16)]
                idx_v[j, pl.ds(q * 16, 16)] = lax.shift_right_logical(v, SHIFT)
            return carry
        lax.fori_loop(0, n_chunks, unpack, 0)

        one = jnp.ones((16,), jnp.float32)
        zero = jnp.zeros((16,), jnp.float32)

        def fill_ones(r, carry):
            for q in range(d // 16):
                ones_v[r, pl.ds(q * 16, 16)] = one
            return carry
        lax.fori_loop(0, CHUNK, fill_ones, 0)

        def fill_zero(r, carry):
            for q in range(d // 16):
                z_v[r, pl.ds(q * 16, 16)] = zero
            return carry
        lax.fori_loop(0, ZROWS, fill_zero, 0)

        base = s * rows_per_tile

        def zero_acc(i, carry):
            pltpu.sync_copy(z_v, acc_sh.at[pl.ds(base + i * ZROWS, ZROWS)])
            return carry
        lax.fori_loop(0, nz, zero_acc, 0)

        plsc.subcore_barrier()

        def body(j, carry):
            pltpu.sync_copy(ones_v, acc_sh.at[idx_v.at[j]], add=True)
            return carry
        lax.fori_loop(0, n_chunks, body, 0)

        plsc.subcore_barrier()
        pltpu.sync_copy(acc_sh.at[pl.ds(base, rows_per_tile)],
                        out_hbm.at[c, pl.ds(base, rows_per_tile)])

    return deg_kernel


def _agg_kernel(n_pad, n_chunks, d):
    rows_per_tile = n_pad // NS
    nz = rows_per_tile // ZROWS
    sec_ch = n_chunks // NSEC

    @functools.partial(
        pl.kernel,
        out_type=jax.ShapeDtypeStruct((NC, n_pad, d), jnp.float32),
        mesh=_sc_mesh(),
        scratch_types=[
            pltpu.VMEM((sec_ch, CHUNK), jnp.int32),
            pltpu.VMEM((sec_ch, CHUNK), jnp.int32),
            pltpu.VMEM((CHUNK, d), jnp.float32),
            pltpu.VMEM((CHUNK, d), jnp.float32),
            pltpu.VMEM((ZROWS, d), jnp.float32),
            pltpu.VMEM_SHARED((n_pad, d), jnp.float32),
            pltpu.SemaphoreType.DMA,
            pltpu.SemaphoreType.DMA,
        ],
    )
    def agg_kernel(x2_hbm, pk_hbm, out_hbm,
                   src_v, dst_v, buf0, buf1, z_v, acc_sh, sem0, sem1):
        c = lax.axis_index("c")
        s = lax.axis_index("s")
        wid = c * NS + s

        zero = jnp.zeros((16,), jnp.float32)

        def fill_zero(r, carry):
            for q in range(d // 16):
                z_v[r, pl.ds(q * 16, 16)] = zero
            return carry
        lax.fori_loop(0, ZROWS, fill_zero, 0)

        base = s * rows_per_tile

        def zero_acc(i, carry):
            pltpu.sync_copy(z_v, acc_sh.at[pl.ds(base + i * ZROWS, ZROWS)])
            return carry
        lax.fori_loop(0, nz, zero_acc, 0)

        plsc.subcore_barrier()

        def section(sec, carry):
            pltpu.sync_copy(pk_hbm.at[wid, pl.ds(sec * sec_ch, sec_ch)], src_v)

            def unpack(j, carry2):
                for q in range(CHUNK // 16):
                    v = src_v[j, pl.ds(q * 16, 16)]
                    dst_v[j, pl.ds(q * 16, 16)] = lax.shift_right_logical(
                        v, SHIFT)
                    src_v[j, pl.ds(q * 16, 16)] = lax.bitwise_and(v, MASK)
                return carry2
            lax.fori_loop(0, sec_ch, unpack, 0)

            pltpu.async_copy(x2_hbm.at[src_v.at[0]], buf0, sem0)

            def body(jj, carry2):
                j = jj * 2
                pltpu.async_copy(x2_hbm.at[src_v.at[j + 1]], buf1, sem1)
                pltpu.make_async_copy(x2_hbm.at[src_v.at[j]], buf0,
                                      sem0).wait()
                pltpu.sync_copy(buf0, acc_sh.at[dst_v.at[j]], add=True)

                @pl.when(jj < (sec_ch // 2) - 1)
                def _():
                    pltpu.async_copy(x2_hbm.at[src_v.at[j + 2]], buf0, sem0)

                pltpu.make_async_copy(x2_hbm.at[src_v.at[j + 1]], buf1,
                                      sem1).wait()
                pltpu.sync_copy(buf1, acc_sh.at[dst_v.at[j + 1]], add=True)
                return carry2
            lax.fori_loop(0, sec_ch // 2, body, 0)
            return carry
        lax.fori_loop(0, NSEC, section, 0)

        plsc.subcore_barrier()
        pltpu.sync_copy(acc_sh.at[pl.ds(base, rows_per_tile)],
                        out_hbm.at[c, pl.ds(base, rows_per_tile)])

    return agg_kernel


def _scale_body(deg_ref, x_ref, o_ref):
    dcol = deg_ref[0][:, :1] + deg_ref[1][:, :1]
    dis = jnp.where(dcol > 0, lax.rsqrt(jnp.where(dcol > 0, dcol, 1.0)), 0.0)
    o_ref[...] = x_ref[...] * dis


def _finish_body(acc_ref, deg_ref, w_ref, b_ref, o_ref):
    dcol = deg_ref[0][:, :1] + deg_ref[1][:, :1]
    dis = jnp.where(dcol > 0, lax.rsqrt(jnp.where(dcol > 0, dcol, 1.0)), 0.0)
    agg = acc_ref[0] + acc_ref[1]
    h = jnp.dot(agg, w_ref[...], preferred_element_type=jnp.float32,
                precision=lax.Precision.HIGHEST)
    o_ref[...] = dis * h + b_ref[...]


def kernel(x, edge_index, W, b):
    n, d_in = x.shape
    d_out = W.shape[1]
    e = edge_index.shape[1]

    n_pad = ((n + 1 + NS * ZROWS - 1) // (NS * ZROWS)) * (NS * ZROWS)
    cm = CHUNK * NSEC * 2
    e_per_tile = ((e + NT * cm - 1) // (NT * cm)) * cm
    n_chunks = e_per_tile // CHUNK
    e_pad = e_per_tile * NT

    src = edge_index[0]
    dst = edge_index[1]
    pk = jnp.bitwise_or(jnp.left_shift(dst, SHIFT), src)
    et = -(-e // NT)
    if e % NT:
        pad0 = NT * et - e
        dums0 = n + (jnp.arange(pad0, dtype=jnp.int32) % (n_pad - n))
        pk = jnp.concatenate([pk, jnp.left_shift(dums0, SHIFT)])
    ppt = e_per_tile - et
    dums = n + (jnp.arange(ppt, dtype=jnp.int32) % (n_pad - n))
    pk_p = jnp.concatenate(
        [pk.reshape(NT, et),
         jnp.broadcast_to(jnp.left_shift(dums, SHIFT)[None, :], (NT, ppt))],
        axis=1,
    ).reshape(NT, n_chunks, CHUNK)

    deg_p = _deg_kernel(n_pad, n_chunks, d_in)(pk_p)

    blk = 1000 if n % 1000 == 0 else min(n, 1024)
    grid = (n + blk - 1) // blk

    x2 = pl.pallas_call(
        _scale_body,
        grid=(grid,),
        in_specs=[
            pl.BlockSpec((NC, blk, d_in), lambda i: (0, i, 0)),
            pl.BlockSpec((blk, d_in), lambda i: (i, 0)),
        ],
        out_specs=pl.BlockSpec((blk, d_in), lambda i: (i, 0)),
        out_shape=jax.ShapeDtypeStruct((n, d_in), jnp.float32),
    )(deg_p, x)

    acc_p = _agg_kernel(n_pad, n_chunks, d_in)(x2, pk_p)

    out = pl.pallas_call(
        _finish_body,
        grid=(grid,),
        in_specs=[
            pl.BlockSpec((NC, blk, d_in), lambda i: (0, i, 0)),
            pl.BlockSpec((NC, blk, d_in), lambda i: (0, i, 0)),
            pl.BlockSpec((d_in, d_out), lambda i: (0, 0)),
            pl.BlockSpec((d_out,), lambda i: (0,)),
        ],
        out_specs=pl.BlockSpec((blk, d_out), lambda i: (i, 0)),
        out_shape=jax.ShapeDtypeStruct((n, d_out), jnp.float32),
    )(acc_p, deg_p, W, b)

    return out
